# baseline (speedup 1.0000x reference)
import functools

import jax
import jax.numpy as jnp
from jax import lax
from jax.experimental import pallas as pl
from jax.experimental.pallas import tpu as pltpu
from jax.experimental.pallas import tpu_sc as plsc

N_VERT = 50000
K = 64
NC, NS = 2, 16
NW = NC * NS
NPASS = 2
RP = 784
NWV = NW * NPASS
NVP = NWV * RP
C = 384
CR = C // 128
MR = 3 * CR
NBUF = 3
F = K // 16
NWP = NWV + 16
FILLR = 16


def _mm_support(x_ref, w_ref, o_ref):
    o_ref[...] = jnp.dot(x_ref[...], w_ref[...], preferred_element_type=jnp.float32)


def _mm_hidden(h_ref, bs_ref, w_ref, o_ref):
    h = jnp.maximum(h_ref[...] + bs_ref[...], 0.0)
    o_ref[...] = jnp.dot(h, w_ref[...], preferred_element_type=jnp.float32)


def _sc_mesh():
    return plsc.VectorSubcoreMesh(core_axis_name="c", subcore_axis_name="s",
                                  num_cores=NC, num_subcores=NS)


_SC_PARAMS = pltpu.CompilerParams(use_tc_tiling_on_sc=False,
                                  needs_layout_passes=False)


def _read_scalar(ref, i):
    return ref[pl.ds(i, 16)][0]


def _lane_bcast(v, l):
    dn = lax.GatherDimensionNumbers(offset_dims=(), collapsed_slice_dims=(0,),
                                    start_index_map=(0,))
    idx = jnp.full((16, 1), l, jnp.int32)
    return lax.gather(v, idx, dn, (1,),
                      mode=lax.GatherScatterMode.PROMISE_IN_BOUNDS)


def _init_acc(acc_sh, fill, shbase, bvec, sem):
    def frow(r, _):
        for f in range(F):
            fill[r, pl.ds(f * 16, 16)] = bvec[f]
        return 0
    lax.fori_loop(0, FILLR, frow, 0)
    for i in range(RP // FILLR):
        pltpu.async_copy(fill, acc_sh.at[pl.ds(shbase + i * FILLR, FILLR)], sem)
    for i in range(RP // FILLR):
        pltpu.make_async_copy(fill, acc_sh.at[pl.ds(shbase, FILLR)], sem).wait()


def _scale_chunk(emeta, msg, dl, db, cb, start, end, base, shbase):
    mrow = db * MR

    def group(gi, _):
        gb = gi * 16
        grow = gi // 8
        gcol = (gi % 8) * 16
        dstv = emeta[mrow + CR + grow, pl.ds(gcol, 16)]
        vv = plsc.bitcast(emeta[mrow + 2 * CR + grow, pl.ds(gcol, 16)],
                          jnp.float32)
        ev = cb + gb + lax.iota(jnp.int32, 16)
        vv = jnp.where((ev >= start) & (ev < end), vv, 0.0)
        dlv = shbase + jnp.clip(dstv - base, 0, RP - 1)
        dl[db * CR + grow, pl.ds(gcol, 16)] = dlv
        for l in range(16):
            bm = _lane_bcast(vv, l)
            er = db * C + gb + l
            for f in range(F):
                sl = pl.ds(f * 16, 16)
                msg[er, sl] = msg[er, sl] * bm
        return 0
    lax.fori_loop(0, C // 16, group, 0)


def _issue_scatter(rows, acc_sh, dl, db, sem):
    for j in range(CR):
        pltpu.async_copy(rows.at[pl.ds(db * C + j * 128, 128)],
                         acc_sh.at[dl.at[db * CR + j]], sem, add=True)


def _drain_scatter(rows, acc_sh, dl, db, sem):
    for j in range(CR):
        pltpu.make_async_copy(rows.at[pl.ds(db * C + j * 128, 128)],
                              acc_sh.at[dl.at[db * CR + j]], sem).wait()


def _spmm_sensor(support, emeta_h, starts, ends, n_sens, nnz2):
    @functools.partial(
        pl.kernel,
        out_type=jax.ShapeDtypeStruct((NVP, K), jnp.float32),
        mesh=_sc_mesh(),
        compiler_params=_SC_PARAMS,
        scratch_types=[
            pltpu.VMEM_SHARED((NS * RP, K), jnp.float32),
            pltpu.VMEM_SHARED((256, K), jnp.float32),
            pltpu.VMEM((NBUF * C, K), jnp.float32),
            pltpu.VMEM((NBUF * MR, 128), jnp.int32),
            pltpu.VMEM((NBUF * CR, 128), jnp.int32),
            pltpu.VMEM((FILLR, K), jnp.float32),
            pltpu.VMEM((NWP,), jnp.int32),
            pltpu.VMEM((NWP,), jnp.int32),
        ] + [pltpu.SemaphoreType.DMA] * (3 * NBUF),
    )
    def k(tab_hbm, emeta_hbm, st_hbm, en_hbm, out_hbm,
          acc_sh, tab_sh, rows, emeta, dl, fill, stv, env,
          ms0, ms1, ms2, gs0, gs1, gs2, ss0, ss1, ss2):
        msem = [ms0, ms1, ms2]
        gsem = [gs0, gs1, gs2]
        ssem = [ss0, ss1, ss2]
        cid = lax.axis_index("c")
        sid = lax.axis_index("s")
        wid = sid * NC + cid
        shbase = sid * RP
        pltpu.sync_copy(st_hbm, stv)
        pltpu.sync_copy(en_hbm, env)

        @pl.when(sid == 0)
        def _():
            pltpu.sync_copy(tab_hbm, tab_sh)
        plsc.subcore_barrier()
        zero = jnp.zeros((16,), jnp.float32)

        def issue_meta(astart, ci, db):
            cb = lax.min(astart + ci * C, nnz2 - C)
            crow = (cb // C) * MR
            pltpu.async_copy(emeta_hbm.at[pl.ds(crow, MR)],
                             emeta.at[pl.ds(db * MR, MR)], msem[db])

        def drain_meta(db):
            pltpu.make_async_copy(emeta_hbm.at[pl.ds(0, MR)],
                                  emeta.at[pl.ds(db * MR, MR)],
                                  msem[db]).wait()

        def issue_gather(db):
            for j in range(CR):
                pltpu.async_copy(tab_sh.at[emeta.at[db * MR + j]],
                                 rows.at[pl.ds(db * C + j * 128, 128)],
                                 gsem[db])

        def drain_gather(db):
            for j in range(CR):
                pltpu.make_async_copy(tab_sh.at[emeta.at[db * MR + j]],
                                      rows.at[pl.ds(db * C + j * 128, 128)],
                                      gsem[db]).wait()

        for p in range(NPASS):
            vw = wid * NPASS + p
            base = vw * RP
            start = _read_scalar(stv, vw)
            end = _read_scalar(env, vw)
            _init_acc(acc_sh, fill, shbase, [zero] * F, ms0)

            astart = (start // C) * C
            nchunks = lax.max((end - astart + C - 1) // C, 1)
            nch3 = (nchunks + 2) // 3

            for db in range(NBUF):
                issue_meta(astart, db, db)
            drain_meta(0)
            issue_gather(0)
            drain_meta(1)
            issue_gather(1)

            def c3loop(c3, _):
                for db in range(NBUF):
                    ci = c3 * 3 + db
                    db2 = (db + 2) % NBUF
                    drain_gather(db)
                    drain_meta(db2)

                    def waits():
                        _drain_scatter(rows, acc_sh, dl, db2, ssem[db2])
                        return 0
                    if db >= 1:
                        waits()
                    else:
                        lax.cond(c3 > 0, waits, lambda: 0)
                    issue_gather(db2)
                    cb = lax.min(astart + ci * C, nnz2 - C)
                    _scale_chunk(emeta, rows, dl, db, cb,
                                 start, end, base, shbase)
                    _issue_scatter(rows, acc_sh, dl, db, ssem[db])
                    issue_meta(astart, ci + 3, db)
                return 0
            lax.fori_loop(0, nch3, c3loop, 0)

            drain_gather(0)
            drain_gather(1)
            drain_meta(2)
            _drain_scatter(rows, acc_sh, dl, 2, ssem[2])
            pltpu.sync_copy(acc_sh.at[pl.ds(shbase, RP)],
                            out_hbm.at[pl.ds(base, RP)])

    return k(support, emeta_h, starts, ends)


def _spmm_vertex(g, emeta_h, b0, starts, ends, nnz2):
    @functools.partial(
        pl.kernel,
        out_type=jax.ShapeDtypeStruct((NVP, K), jnp.float32),
        mesh=_sc_mesh(),
        compiler_params=_SC_PARAMS,
        scratch_types=[
            pltpu.VMEM_SHARED((NS * RP, K), jnp.float32),
            pltpu.VMEM((NBUF * C, K), jnp.float32),
            pltpu.VMEM((NBUF * MR, 128), jnp.int32),
            pltpu.VMEM((NBUF * CR, 128), jnp.int32),
            pltpu.VMEM((K,), jnp.float32),
            pltpu.VMEM((FILLR, K), jnp.float32),
            pltpu.VMEM((NWP,), jnp.int32),
            pltpu.VMEM((NWP,), jnp.int32),
        ] + [pltpu.SemaphoreType.DMA] * (3 * NBUF),
    )
    def k(g_hbm, emeta_hbm, b0_hbm, st_hbm, en_hbm, out_hbm,
          acc_sh, rows, emeta, dl, b0v, fill, stv, env,
          ms0, ms1, ms2, gs0, gs1, gs2, ss0, ss1, ss2):
        msem = [ms0, ms1, ms2]
        gsem = [gs0, gs1, gs2]
        ssem = [ss0, ss1, ss2]
        cid = lax.axis_index("c")
        sid = lax.axis_index("s")
        wid = sid * NC + cid
        shbase = sid * RP
        pltpu.sync_copy(st_hbm, stv)
        pltpu.sync_copy(en_hbm, env)
        pltpu.sync_copy(b0_hbm, b0v)
        bvec = [b0v[pl.ds(f * 16, 16)] for f in range(F)]

        def issue_meta(astart, ci, db):
            cb = lax.min(astart + ci * C, nnz2 - C)
            crow = (cb // C) * MR
            pltpu.async_copy(emeta_hbm.at[pl.ds(crow, MR)],
                             emeta.at[pl.ds(db * MR, MR)], msem[db])

        def drain_meta(db):
            pltpu.make_async_copy(emeta_hbm.at[pl.ds(0, MR)],
                                  emeta.at[pl.ds(db * MR, MR)],
                                  msem[db]).wait()

        def issue_gather(db):
            for j in range(CR):
                pltpu.async_copy(g_hbm.at[emeta.at[db * MR + j]],
                                 rows.at[pl.ds(db * C + j * 128, 128)],
                                 gsem[db])

        def drain_gather(db):
            for j in range(CR):
                pltpu.make_async_copy(g_hbm.at[emeta.at[db * MR + j]],
                                      rows.at[pl.ds(db * C + j * 128, 128)],
                                      gsem[db]).wait()

        for p in range(NPASS):
            vw = wid * NPASS + p
            base = vw * RP
            start = _read_scalar(stv, vw)
            end = _read_scalar(env, vw)
            _init_acc(acc_sh, fill, shbase, bvec, ms0)

            astart = (start // C) * C
            nchunks = lax.max((end - astart + C - 1) // C, 1)
            nch3 = (nchunks + 2) // 3

            for db in range(NBUF):
                issue_meta(astart, db, db)
            drain_meta(0)
            issue_gather(0)

            def c3loop(c3, _):
                for db in range(NBUF):
                    ci = c3 * 3 + db
                    db1 = (db + 1) % NBUF
                    drain_gather(db)
                    drain_meta(db1)

                    def waits():
                        _drain_scatter(rows, acc_sh, dl, db1, ssem[db1])
                        return 0
                    if db == 2:
                        waits()
                    else:
                        lax.cond(c3 > 0, waits, lambda: 0)
                    issue_gather(db1)
                    cb = lax.min(astart + ci * C, nnz2 - C)
                    _scale_chunk(emeta, rows, dl, db, cb,
                                 start, end, base, shbase)
                    _issue_scatter(rows, acc_sh, dl, db, ssem[db])
                    issue_meta(astart, ci + 3, db)
                return 0
            lax.fori_loop(0, nch3, c3loop, 0)

            drain_gather(0)
            drain_meta(1)
            drain_meta(2)
            _drain_scatter(rows, acc_sh, dl, 1, ssem[1])
            _drain_scatter(rows, acc_sh, dl, 2, ssem[2])
            pltpu.sync_copy(acc_sh.at[pl.ds(shbase, RP)],
                            out_hbm.at[pl.ds(base, RP)])

    return k(g, emeta_h, b0, starts, ends)


def _pack_edges(src, dst, vals):
    nnz = src.shape[0]
    nnz2 = (((nnz + C - 1) // C) + NBUF) * C
    pad = nnz2 - nnz
    src = jnp.pad(src, (0, pad))
    dst = jnp.pad(dst, (0, pad), constant_values=NVP - 1)
    vals = jnp.pad(vals, (0, pad))
    vals_i = lax.bitcast_convert_type(vals, jnp.int32)
    emeta = jnp.concatenate(
        [src.reshape(-1, CR, 128), dst.reshape(-1, CR, 128),
         vals_i.reshape(-1, CR, 128)], axis=1).reshape(-1, 128)
    bounds = (jnp.arange(NWV, dtype=jnp.int32) * RP).astype(dst.dtype)
    starts = jnp.searchsorted(dst, bounds, side="left").astype(jnp.int32)
    ends = jnp.concatenate(
        [starts[1:], jnp.array([nnz2], dtype=jnp.int32)])
    ends = jnp.minimum(ends, nnz2 - C)
    starts = jnp.pad(starts, (0, NWP - NWV))
    ends = jnp.pad(ends, (0, NWP - NWV))
    return emeta, starts, ends, nnz2


def kernel(x, vals_s, vals_v, W_s, b_s, W0, b0, src_s, dst_s, src_v, dst_v):
    n_sens, k = x.shape
    l0 = W_s.shape[1]

    support = pl.pallas_call(
        _mm_support,
        out_shape=jax.ShapeDtypeStruct((n_sens, l0), jnp.float32),
    )(x, W_s)

    emeta_s, starts_s, ends_s, nnz2_s = _pack_edges(src_s, dst_s, vals_s)
    hpre = _spmm_sensor(support, emeta_s, starts_s, ends_s, n_sens, nnz2_s)

    BLK = 1568
    g = pl.pallas_call(
        _mm_hidden,
        grid=(NVP // BLK,),
        in_specs=[
            pl.BlockSpec((BLK, l0), lambda i: (i, 0)),
            pl.BlockSpec((1, l0), lambda i: (0, 0)),
            pl.BlockSpec((l0, k), lambda i: (0, 0)),
        ],
        out_specs=pl.BlockSpec((BLK, k), lambda i: (i, 0)),
        out_shape=jax.ShapeDtypeStruct((NVP, k), jnp.float32),
    )(hpre, b_s.reshape(1, l0), W0)

    emeta_v, starts_v, ends_v, nnz2_v = _pack_edges(src_v, dst_v, vals_v)
    out = _spmm_vertex(g, emeta_v, b0, starts_v, ends_v, nnz2_v)
    return out[:N_VERT]


# speedup vs baseline: 1.0701x; 1.0701x over previous
import functools

import jax
import jax.numpy as jnp
from jax import lax
from jax.experimental import pallas as pl
from jax.experimental.pallas import tpu as pltpu
from jax.experimental.pallas import tpu_sc as plsc

N_VERT = 50000
K = 64
NC, NS = 2, 16
NW = NC * NS
NPASS = 2
RP = 784
NWV = NW * NPASS
NVP = NWV * RP
C = 256
CR = C // 128
MR = 3 * CR
NBUF = 3
F = K // 16
NWP = NWV + 16
FILLR = 56


def _mm_support(x_ref, w_ref, o_ref):
    o_ref[...] = jnp.dot(x_ref[...], w_ref[...], preferred_element_type=jnp.float32)


def _mm_hidden(h_ref, bs_ref, w_ref, o_ref):
    h = jnp.maximum(h_ref[...] + bs_ref[...], 0.0)
    o_ref[...] = jnp.dot(h, w_ref[...], preferred_element_type=jnp.float32)


def _sc_mesh():
    return plsc.VectorSubcoreMesh(core_axis_name="c", subcore_axis_name="s",
                                  num_cores=NC, num_subcores=NS)


_SC_PARAMS = pltpu.CompilerParams(use_tc_tiling_on_sc=False,
                                  needs_layout_passes=False)


def _read_scalar(ref, i):
    return ref[pl.ds(i, 16)][0]


def _lane_bcast(v, l):
    dn = lax.GatherDimensionNumbers(offset_dims=(), collapsed_slice_dims=(0,),
                                    start_index_map=(0,))
    idx = jnp.full((16, 1), l, jnp.int32)
    return lax.gather(v, idx, dn, (1,),
                      mode=lax.GatherScatterMode.PROMISE_IN_BOUNDS)


def _init_acc(acc_sh, fill, shbase, bvec):
    def frow(r, _):
        for f in range(F):
            fill[r, pl.ds(f * 16, 16)] = bvec[f]
        return 0
    lax.fori_loop(0, FILLR, frow, 0)
    for i in range(RP // FILLR):
        pltpu.sync_copy(fill, acc_sh.at[pl.ds(shbase + i * FILLR, FILLR)])


def _scale_chunk(emeta, msg, dl, db, cb, start, end, base, shbase):
    mrow = db * MR

    def group(gi, _):
        gb = gi * 16
        grow = gi // 8
        gcol = (gi % 8) * 16
        dstv = emeta[mrow + CR + grow, pl.ds(gcol, 16)]
        vv = plsc.bitcast(emeta[mrow + 2 * CR + grow, pl.ds(gcol, 16)],
                          jnp.float32)
        ev = cb + gb + lax.iota(jnp.int32, 16)
        vv = jnp.where((ev >= start) & (ev < end), vv, 0.0)
        dlv = shbase + jnp.clip(dstv - base, 0, RP - 1)
        dl[db * CR + grow, pl.ds(gcol, 16)] = dlv
        for l in range(16):
            bm = _lane_bcast(vv, l)
            er = db * C + gb + l
            for f in range(F):
                sl = pl.ds(f * 16, 16)
                msg[er, sl] = msg[er, sl] * bm
        return 0
    lax.fori_loop(0, C // 16, group, 0)


def _issue_scatter(rows, acc_sh, dl, db, sem):
    for j in range(CR):
        pltpu.async_copy(rows.at[pl.ds(db * C + j * 128, 128)],
                         acc_sh.at[dl.at[db * CR + j]], sem, add=True)


def _drain_scatter(rows, acc_sh, dl, db, sem):
    for j in range(CR):
        pltpu.make_async_copy(rows.at[pl.ds(db * C + j * 128, 128)],
                              acc_sh.at[dl.at[db * CR + j]], sem).wait()


def _spmm_sensor(support, emeta_h, starts, ends, n_sens, nnz2):
    @functools.partial(
        pl.kernel,
        out_type=jax.ShapeDtypeStruct((NVP, K), jnp.float32),
        mesh=_sc_mesh(),
        compiler_params=_SC_PARAMS,
        scratch_types=[
            pltpu.VMEM_SHARED((NS * RP, K), jnp.float32),
            pltpu.VMEM_SHARED((256, K), jnp.float32),
            pltpu.VMEM((NBUF * C, K), jnp.float32),
            pltpu.VMEM((NBUF * MR, 128), jnp.int32),
            pltpu.VMEM((NBUF * CR, 128), jnp.int32),
            pltpu.VMEM((FILLR, K), jnp.float32),
            pltpu.VMEM((NWP,), jnp.int32),
            pltpu.VMEM((NWP,), jnp.int32),
        ] + [pltpu.SemaphoreType.DMA] * (3 * NBUF),
    )
    def k(tab_hbm, emeta_hbm, st_hbm, en_hbm, out_hbm,
          acc_sh, tab_sh, rows, emeta, dl, fill, stv, env,
          ms0, ms1, ms2, gs0, gs1, gs2, ss0, ss1, ss2):
        msem = [ms0, ms1, ms2]
        gsem = [gs0, gs1, gs2]
        ssem = [ss0, ss1, ss2]
        cid = lax.axis_index("c")
        sid = lax.axis_index("s")
        wid = sid * NC + cid
        shbase = sid * RP
        pltpu.sync_copy(st_hbm, stv)
        pltpu.sync_copy(en_hbm, env)

        @pl.when(sid == 0)
        def _():
            pltpu.sync_copy(tab_hbm, tab_sh)
        plsc.subcore_barrier()
        zero = jnp.zeros((16,), jnp.float32)

        def issue_meta(astart, ci, db):
            cb = lax.min(astart + ci * C, nnz2 - C)
            crow = (cb // C) * MR
            pltpu.async_copy(emeta_hbm.at[pl.ds(crow, MR)],
                             emeta.at[pl.ds(db * MR, MR)], msem[db])

        def drain_meta(db):
            pltpu.make_async_copy(emeta_hbm.at[pl.ds(0, MR)],
                                  emeta.at[pl.ds(db * MR, MR)],
                                  msem[db]).wait()

        def issue_gather(db):
            for j in range(CR):
                pltpu.async_copy(tab_sh.at[emeta.at[db * MR + j]],
                                 rows.at[pl.ds(db * C + j * 128, 128)],
                                 gsem[db])

        def drain_gather(db):
            for j in range(CR):
                pltpu.make_async_copy(tab_sh.at[emeta.at[db * MR + j]],
                                      rows.at[pl.ds(db * C + j * 128, 128)],
                                      gsem[db]).wait()

        for p in range(NPASS):
            vw = wid * NPASS + p
            base = vw * RP
            start = _read_scalar(stv, vw)
            end = _read_scalar(env, vw)
            _init_acc(acc_sh, fill, shbase, [zero] * F)

            astart = (start // C) * C
            nchunks = lax.max((end - astart + C - 1) // C, 1)
            nch3 = (nchunks + 2) // 3

            for db in range(NBUF):
                issue_meta(astart, db, db)
            drain_meta(0)
            issue_gather(0)
            drain_meta(1)
            issue_gather(1)

            def c3loop(c3, _):
                for db in range(NBUF):
                    ci = c3 * 3 + db
                    db2 = (db + 2) % NBUF
                    drain_gather(db)
                    drain_meta(db2)

                    def waits():
                        _drain_scatter(rows, acc_sh, dl, db2, ssem[db2])
                        return 0
                    if db >= 1:
                        waits()
                    else:
                        lax.cond(c3 > 0, waits, lambda: 0)
                    issue_gather(db2)
                    cb = lax.min(astart + ci * C, nnz2 - C)
                    _scale_chunk(emeta, rows, dl, db, cb,
                                 start, end, base, shbase)
                    _issue_scatter(rows, acc_sh, dl, db, ssem[db])
                    issue_meta(astart, ci + 3, db)
                return 0
            lax.fori_loop(0, nch3, c3loop, 0)

            drain_gather(0)
            drain_gather(1)
            drain_meta(2)
            _drain_scatter(rows, acc_sh, dl, 2, ssem[2])
            pltpu.sync_copy(acc_sh.at[pl.ds(shbase, RP)],
                            out_hbm.at[pl.ds(base, RP)])

    return k(support, emeta_h, starts, ends)


def _spmm_vertex(g, emeta_h, b0, starts, ends, nnz2):
    @functools.partial(
        pl.kernel,
        out_type=jax.ShapeDtypeStruct((NVP, K), jnp.float32),
        mesh=_sc_mesh(),
        compiler_params=_SC_PARAMS,
        scratch_types=[
            pltpu.VMEM_SHARED((NS * RP, K), jnp.float32),
            pltpu.VMEM((NBUF * C, K), jnp.float32),
            pltpu.VMEM((NBUF * MR, 128), jnp.int32),
            pltpu.VMEM((NBUF * CR, 128), jnp.int32),
            pltpu.VMEM((K,), jnp.float32),
            pltpu.VMEM((FILLR, K), jnp.float32),
            pltpu.VMEM((NWP,), jnp.int32),
            pltpu.VMEM((NWP,), jnp.int32),
        ] + [pltpu.SemaphoreType.DMA] * (3 * NBUF),
    )
    def k(g_hbm, emeta_hbm, b0_hbm, st_hbm, en_hbm, out_hbm,
          acc_sh, rows, emeta, dl, b0v, fill, stv, env,
          ms0, ms1, ms2, gs0, gs1, gs2, ss0, ss1, ss2):
        msem = [ms0, ms1, ms2]
        gsem = [gs0, gs1, gs2]
        ssem = [ss0, ss1, ss2]
        cid = lax.axis_index("c")
        sid = lax.axis_index("s")
        wid = sid * NC + cid
        shbase = sid * RP
        pltpu.sync_copy(st_hbm, stv)
        pltpu.sync_copy(en_hbm, env)
        pltpu.sync_copy(b0_hbm, b0v)
        bvec = [b0v[pl.ds(f * 16, 16)] for f in range(F)]

        def issue_meta(astart, ci, db):
            cb = lax.min(astart + ci * C, nnz2 - C)
            crow = (cb // C) * MR
            pltpu.async_copy(emeta_hbm.at[pl.ds(crow, MR)],
                             emeta.at[pl.ds(db * MR, MR)], msem[db])

        def drain_meta(db):
            pltpu.make_async_copy(emeta_hbm.at[pl.ds(0, MR)],
                                  emeta.at[pl.ds(db * MR, MR)],
                                  msem[db]).wait()

        def issue_gather(db):
            for j in range(CR):
                pltpu.async_copy(g_hbm.at[emeta.at[db * MR + j]],
                                 rows.at[pl.ds(db * C + j * 128, 128)],
                                 gsem[db])

        def drain_gather(db):
            for j in range(CR):
                pltpu.make_async_copy(g_hbm.at[emeta.at[db * MR + j]],
                                      rows.at[pl.ds(db * C + j * 128, 128)],
                                      gsem[db]).wait()

        for p in range(NPASS):
            vw = wid * NPASS + p
            base = vw * RP
            start = _read_scalar(stv, vw)
            end = _read_scalar(env, vw)
            _init_acc(acc_sh, fill, shbase, bvec)

            astart = (start // C) * C
            nchunks = lax.max((end - astart + C - 1) // C, 1)
            nch3 = (nchunks + 2) // 3

            for db in range(NBUF):
                issue_meta(astart, db, db)
            drain_meta(0)
            issue_gather(0)

            def c3loop(c3, _):
                for db in range(NBUF):
                    ci = c3 * 3 + db
                    db1 = (db + 1) % NBUF
                    drain_gather(db)
                    drain_meta(db1)

                    def waits():
                        _drain_scatter(rows, acc_sh, dl, db1, ssem[db1])
                        return 0
                    if db == 2:
                        waits()
                    else:
                        lax.cond(c3 > 0, waits, lambda: 0)
                    issue_gather(db1)
                    cb = lax.min(astart + ci * C, nnz2 - C)
                    _scale_chunk(emeta, rows, dl, db, cb,
                                 start, end, base, shbase)
                    _issue_scatter(rows, acc_sh, dl, db, ssem[db])
                    issue_meta(astart, ci + 3, db)
                return 0
            lax.fori_loop(0, nch3, c3loop, 0)

            drain_gather(0)
            drain_meta(1)
            drain_meta(2)
            _drain_scatter(rows, acc_sh, dl, 1, ssem[1])
            _drain_scatter(rows, acc_sh, dl, 2, ssem[2])
            pltpu.sync_copy(acc_sh.at[pl.ds(shbase, RP)],
                            out_hbm.at[pl.ds(base, RP)])

    return k(g, emeta_h, b0, starts, ends)


def _pack_edges(src, dst, vals):
    nnz = src.shape[0]
    nnz2 = (((nnz + C - 1) // C) + NBUF) * C
    pad = nnz2 - nnz
    src = jnp.pad(src, (0, pad))
    dst = jnp.pad(dst, (0, pad), constant_values=NVP - 1)
    vals = jnp.pad(vals, (0, pad))
    vals_i = lax.bitcast_convert_type(vals, jnp.int32)
    emeta = jnp.concatenate(
        [src.reshape(-1, CR, 128), dst.reshape(-1, CR, 128),
         vals_i.reshape(-1, CR, 128)], axis=1).reshape(-1, 128)
    bounds = (jnp.arange(NWV, dtype=jnp.int32) * RP).astype(dst.dtype)
    starts = jnp.searchsorted(dst, bounds, side="left").astype(jnp.int32)
    ends = jnp.concatenate(
        [starts[1:], jnp.array([nnz2], dtype=jnp.int32)])
    ends = jnp.minimum(ends, nnz2 - C)
    starts = jnp.pad(starts, (0, NWP - NWV))
    ends = jnp.pad(ends, (0, NWP - NWV))
    return emeta, starts, ends, nnz2


def kernel(x, vals_s, vals_v, W_s, b_s, W0, b0, src_s, dst_s, src_v, dst_v):
    n_sens, k = x.shape
    l0 = W_s.shape[1]

    support = pl.pallas_call(
        _mm_support,
        out_shape=jax.ShapeDtypeStruct((n_sens, l0), jnp.float32),
    )(x, W_s)

    emeta_s, starts_s, ends_s, nnz2_s = _pack_edges(src_s, dst_s, vals_s)
    hpre = _spmm_sensor(support, emeta_s, starts_s, ends_s, n_sens, nnz2_s)

    BLK = 1568
    g = pl.pallas_call(
        _mm_hidden,
        grid=(NVP // BLK,),
        in_specs=[
            pl.BlockSpec((BLK, l0), lambda i: (i, 0)),
            pl.BlockSpec((1, l0), lambda i: (0, 0)),
            pl.BlockSpec((l0, k), lambda i: (0, 0)),
        ],
        out_specs=pl.BlockSpec((BLK, k), lambda i: (i, 0)),
        out_shape=jax.ShapeDtypeStruct((NVP, k), jnp.float32),
    )(hpre, b_s.reshape(1, l0), W0)

    emeta_v, starts_v, ends_v, nnz2_v = _pack_edges(src_v, dst_v, vals_v)
    out = _spmm_vertex(g, emeta_v, b0, starts_v, ends_v, nnz2_v)
    return out[:N_VERT]
